# baseline (speedup 1.0000x reference)
import functools

import jax
import jax.numpy as jnp
from jax.experimental import pallas as pl

BN = 2000
NB = 100000 // BN


def _sample_kernel(w_ref, s_ref, bg_ref, max_ref, arg_ref):
    i = pl.program_id(0)
    y = jax.lax.dot_general(
        w_ref[...], s_ref[...], (((1,), (0,)), ((), ())),
        preferred_element_type=jnp.float32,
        precision=jax.lax.Precision.HIGHEST,
    ) + bg_ref[...]
    m = jnp.max(y, axis=0, keepdims=True)
    rows = jax.lax.broadcasted_iota(jnp.int32, (BN, 8), 0)
    a = jnp.min(jnp.where(y == m, rows, BN), axis=0, keepdims=True)

    @pl.when(i == 0)
    def _init():
        max_ref[...] = m
        arg_ref[...] = a

    @pl.when(i != 0)
    def _merge():
        better = m > max_ref[...]
        arg_ref[...] = jnp.where(better, a + i * BN, arg_ref[...])
        max_ref[...] = jnp.maximum(m, max_ref[...])


@functools.partial(jax.jit, static_argnames=())
def kernel(state, W, b):
    g = jax.random.gumbel(jax.random.key(42), (8, 100000), jnp.float32)
    bg = g.T + b[:, None]
    sT = state.T

    _, arg = pl.pallas_call(
        _sample_kernel,
        grid=(NB,),
        in_specs=[
            pl.BlockSpec((BN, 1024), lambda i: (i, 0)),
            pl.BlockSpec((1024, 8), lambda i: (0, 0)),
            pl.BlockSpec((BN, 8), lambda i: (i, 0)),
        ],
        out_specs=[
            pl.BlockSpec((1, 8), lambda i: (0, 0)),
            pl.BlockSpec((1, 8), lambda i: (0, 0)),
        ],
        out_shape=[
            jax.ShapeDtypeStruct((1, 8), jnp.float32),
            jax.ShapeDtypeStruct((1, 8), jnp.int32),
        ],
    )(W, sT, bg)
    return arg.reshape(8, 1).astype(jnp.int64)


# speedup vs baseline: 1.5896x; 1.5896x over previous
import functools

import jax
import jax.numpy as jnp
from jax.experimental import pallas as pl

BN = 2000
NB = 100000 // BN


def _sample_kernel(w_ref, s_ref, bg_ref, max_ref, arg_ref):
    i = pl.program_id(0)
    w16 = w_ref[...].astype(jnp.bfloat16)
    yp = jax.lax.dot_general(
        w16, s_ref[...], (((1,), (0,)), ((), ())),
        preferred_element_type=jnp.float32,
    )
    y = yp[:, 0:8] + yp[:, 8:16] + yp[:, 16:24] + bg_ref[...]
    m = jnp.max(y, axis=0, keepdims=True)
    rows = jax.lax.broadcasted_iota(jnp.int32, (BN, 8), 0)
    a = jnp.min(jnp.where(y == m, rows, BN), axis=0, keepdims=True)

    @pl.when(i == 0)
    def _init():
        max_ref[...] = m
        arg_ref[...] = a

    @pl.when(i != 0)
    def _merge():
        better = m > max_ref[...]
        arg_ref[...] = jnp.where(better, a + i * BN, arg_ref[...])
        max_ref[...] = jnp.maximum(m, max_ref[...])


@functools.partial(jax.jit, static_argnames=())
def kernel(state, W, b):
    g = jax.random.gumbel(jax.random.key(42), (8, 100000), jnp.float32)
    bg = g.T + b[:, None]
    sT = state.T
    hi = sT.astype(jnp.bfloat16)
    r1 = sT - hi.astype(jnp.float32)
    mid = r1.astype(jnp.bfloat16)
    lo = (r1 - mid.astype(jnp.float32)).astype(jnp.bfloat16)
    rhs = jnp.concatenate([hi, mid, lo], axis=1)

    _, arg = pl.pallas_call(
        _sample_kernel,
        grid=(NB,),
        in_specs=[
            pl.BlockSpec((BN, 1024), lambda i: (i, 0)),
            pl.BlockSpec((1024, 24), lambda i: (0, 0)),
            pl.BlockSpec((BN, 8), lambda i: (i, 0)),
        ],
        out_specs=[
            pl.BlockSpec((1, 8), lambda i: (0, 0)),
            pl.BlockSpec((1, 8), lambda i: (0, 0)),
        ],
        out_shape=[
            jax.ShapeDtypeStruct((1, 8), jnp.float32),
            jax.ShapeDtypeStruct((1, 8), jnp.int32),
        ],
    )(W, rhs, bg)
    return arg.reshape(8, 1).astype(jnp.int64)


# speedup vs baseline: 2.9430x; 1.8515x over previous
import functools

import jax
import jax.numpy as jnp
from jax.experimental import pallas as pl

BN = 2000
NB = 100000 // BN


def _sample_kernel(w_ref, s_ref, g_ref, b_ref, max_ref, arg_ref):
    i = pl.program_id(0)
    w16 = w_ref[...].astype(jnp.bfloat16)
    yp = jax.lax.dot_general(
        w16, s_ref[...], (((1,), (0,)), ((), ())),
        preferred_element_type=jnp.float32,
    )
    gt = g_ref[0].T
    y = yp[:, 0:8] + yp[:, 8:16] + yp[:, 16:24] + gt + b_ref[...]
    m = jnp.max(y, axis=0, keepdims=True)
    rows = jax.lax.broadcasted_iota(jnp.int32, (BN, 8), 0)
    a = jnp.min(jnp.where(y == m, rows, BN), axis=0, keepdims=True)

    @pl.when(i == 0)
    def _init():
        max_ref[...] = m
        arg_ref[...] = a

    @pl.when(i != 0)
    def _merge():
        better = m > max_ref[...]
        arg_ref[...] = jnp.where(better, a + i * BN, arg_ref[...])
        max_ref[...] = jnp.maximum(m, max_ref[...])


@functools.partial(jax.jit, static_argnames=())
def kernel(state, W, b):
    g = jax.random.gumbel(jax.random.key(42), (8, 100000), jnp.float32)
    g3 = jnp.swapaxes(g.reshape(8, NB, BN), 0, 1)
    sT = state.T
    hi = sT.astype(jnp.bfloat16)
    r1 = sT - hi.astype(jnp.float32)
    mid = r1.astype(jnp.bfloat16)
    lo = (r1 - mid.astype(jnp.float32)).astype(jnp.bfloat16)
    rhs = jnp.concatenate([hi, mid, lo], axis=1)

    _, arg = pl.pallas_call(
        _sample_kernel,
        grid=(NB,),
        in_specs=[
            pl.BlockSpec((BN, 1024), lambda i: (i, 0)),
            pl.BlockSpec((1024, 24), lambda i: (0, 0)),
            pl.BlockSpec((1, 8, BN), lambda i: (i, 0, 0)),
            pl.BlockSpec((BN, 1), lambda i: (i, 0)),
        ],
        out_specs=[
            pl.BlockSpec((1, 8), lambda i: (0, 0)),
            pl.BlockSpec((1, 8), lambda i: (0, 0)),
        ],
        out_shape=[
            jax.ShapeDtypeStruct((1, 8), jnp.float32),
            jax.ShapeDtypeStruct((1, 8), jnp.int32),
        ],
    )(W, rhs, g3, b[:, None])
    return arg.reshape(8, 1).astype(jnp.int64)
